# baseline (speedup 1.0000x reference)
import functools

import jax
import jax.numpy as jnp
from jax import lax
from jax.experimental import pallas as pl
from jax.experimental.pallas import tpu as pltpu
from jax.experimental.pallas import tpu_sc as plsc

_BATCH = 16384
_L = 16

_LOG1P_COEFS = (
    0.9999998102178485, -0.4999744938483586, 0.3327617657151469,
    -0.24499611724550963, 0.17757023992299661, -0.10785367917171329,
    0.04421419233802388, -0.008574676204766396,
)


def _log1p_poly(u):
    acc = jnp.float32(_LOG1P_COEFS[-1])
    for c in _LOG1P_COEFS[-2::-1]:
        acc = acc * u + jnp.float32(c)
    return u * acc


def _softplus(x):
    return jnp.maximum(x, 0.0) + _log1p_poly(jnp.exp(-jnp.abs(x)))


def _flatten_padded(w):
    n = w.shape[0]
    n_pad = -n % 1024
    if n_pad:
        w = jnp.pad(w, ((0, n_pad), (0, 0)))
    return w.reshape(-1)


def _make_item_kernel(num_cores, b_per_w):
    mesh = plsc.VectorSubcoreMesh(core_axis_name="c", subcore_axis_name="s")
    out = jax.ShapeDtypeStruct((_BATCH,), jnp.float32)

    @functools.partial(
        pl.kernel,
        mesh=mesh,
        out_type=(out, out, out),
        scratch_types=[
            pltpu.VMEM((b_per_w,), jnp.int32),
            pltpu.VMEM((b_per_w,), jnp.float32),
            pltpu.VMEM((b_per_w,), jnp.float32),
            pltpu.VMEM((b_per_w,), jnp.float32),
            pltpu.VMEM((b_per_w,), jnp.float32),
            pltpu.VMEM((b_per_w,), jnp.float32),
            pltpu.VMEM((b_per_w,), jnp.float32),
            pltpu.SemaphoreType.DMA,
        ],
    )
    def k(item_hbm, a_hbm, b_hbm, c_hbm, p_hbm, q_hbm, cs_hbm,
          i_idx, a_v, b_v, c_v, p_v, q_v, cs_v, sem):
        wid = lax.axis_index("s") * num_cores + lax.axis_index("c")
        base = wid * b_per_w
        sl_out = pl.ds(base, b_per_w)

        pltpu.sync_copy(item_hbm.at[sl_out], i_idx)
        g_a = pltpu.make_async_copy(a_hbm.at[i_idx], a_v, sem)
        g_b = pltpu.make_async_copy(b_hbm.at[i_idx], b_v, sem)
        g_c = pltpu.make_async_copy(c_hbm.at[i_idx], c_v, sem)
        g_a.start()
        g_b.start()
        g_c.start()
        g_a.wait()
        g_b.wait()
        g_c.wait()

        def body_a(i, carry):
            sl = pl.ds(i * _L, _L)
            p = -1.73 * _softplus(a_v[sl])
            p_v[sl] = p
            q_v[sl] = 1e-08 - p * b_v[sl]
            cs_v[sl] = 1.0 / (1.0 + jnp.exp(-c_v[sl]))
            return carry

        lax.fori_loop(0, b_per_w // _L, body_a, 0, unroll=4)

        cp_p = pltpu.make_async_copy(p_v, p_hbm.at[sl_out], sem)
        cp_q = pltpu.make_async_copy(q_v, q_hbm.at[sl_out], sem)
        cp_c = pltpu.make_async_copy(cs_v, cs_hbm.at[sl_out], sem)
        cp_p.start()
        cp_q.start()
        cp_c.start()
        cp_p.wait()
        cp_q.wait()
        cp_c.wait()

    return k


def _make_theta_kernel(num_cores, b_per_w):
    mesh = plsc.VectorSubcoreMesh(core_axis_name="c", subcore_axis_name="s")

    @functools.partial(
        pl.kernel,
        mesh=mesh,
        out_type=jax.ShapeDtypeStruct((_BATCH,), jnp.float32),
        scratch_types=[
            pltpu.VMEM((b_per_w,), jnp.int32),
            pltpu.VMEM((b_per_w,), jnp.float32),
            pltpu.VMEM((b_per_w,), jnp.float32),
            pltpu.VMEM((b_per_w,), jnp.float32),
            pltpu.VMEM((b_per_w,), jnp.float32),
            pltpu.VMEM((b_per_w,), jnp.float32),
            pltpu.SemaphoreType.DMA,
        ],
    )
    def k(user_hbm, theta_hbm, p_hbm, q_hbm, cs_hbm, out_hbm,
          u_idx, th_v, p_v, q_v, cs_v, o_v, sem):
        wid = lax.axis_index("s") * num_cores + lax.axis_index("c")
        base = wid * b_per_w
        sl_out = pl.ds(base, b_per_w)

        cp_u = pltpu.make_async_copy(user_hbm.at[sl_out], u_idx, sem)
        cp_p = pltpu.make_async_copy(p_hbm.at[sl_out], p_v, sem)
        cp_q = pltpu.make_async_copy(q_hbm.at[sl_out], q_v, sem)
        cp_c = pltpu.make_async_copy(cs_hbm.at[sl_out], cs_v, sem)
        cp_u.start()
        cp_p.start()
        cp_q.start()
        cp_c.start()
        cp_u.wait()
        g_th = pltpu.make_async_copy(theta_hbm.at[u_idx], th_v, sem)
        g_th.start()
        cp_p.wait()
        cp_q.wait()
        cp_c.wait()
        g_th.wait()

        def body_t(i, carry):
            sl = pl.ds(i * _L, _L)
            cs = cs_v[sl]
            z = p_v[sl] * th_v[sl] + q_v[sl]
            o_v[sl] = cs + (1.0 - cs) / (1.0 + jnp.exp(z))
            return carry

        lax.fori_loop(0, b_per_w // _L, body_t, 0, unroll=4)

        pltpu.sync_copy(o_v, out_hbm.at[sl_out])

    return k


def kernel(user, item, theta_w, a_w, b_w, c_w):
    info = plsc.get_sparse_core_info()
    num_workers = info.num_cores * info.num_subcores
    b_per_w = _BATCH // num_workers
    k_item = _make_item_kernel(info.num_cores, b_per_w)
    k_theta = _make_theta_kernel(info.num_cores, b_per_w)
    p, q, cs = k_item(
        item.astype(jnp.int32),
        _flatten_padded(a_w),
        _flatten_padded(b_w),
        _flatten_padded(c_w),
    )
    return k_theta(
        user.astype(jnp.int32),
        _flatten_padded(theta_w),
        p, q, cs,
    )


# speedup vs baseline: 1.0116x; 1.0116x over previous
import functools

import jax
import jax.numpy as jnp
from jax import lax
from jax.experimental import pallas as pl
from jax.experimental.pallas import tpu as pltpu
from jax.experimental.pallas import tpu_sc as plsc

_BATCH = 16384
_L = 16

_LOG1P_COEFS = (
    0.9999998102178485, -0.4999744938483586, 0.3327617657151469,
    -0.24499611724550963, 0.17757023992299661, -0.10785367917171329,
    0.04421419233802388, -0.008574676204766396,
)


def _log1p_poly(u):
    acc = jnp.float32(_LOG1P_COEFS[-1])
    for c in _LOG1P_COEFS[-2::-1]:
        acc = acc * u + jnp.float32(c)
    return u * acc


def _softplus(x):
    return jnp.maximum(x, 0.0) + _log1p_poly(jnp.exp(-jnp.abs(x)))


def _flatten_padded(w):
    n = w.shape[0]
    n_pad = -n % 1024
    if n_pad:
        w = jnp.pad(w, ((0, n_pad), (0, 0)))
    return w.reshape(-1)


def _make_item_kernel(num_cores, b_per_w):
    mesh = plsc.VectorSubcoreMesh(core_axis_name="c", subcore_axis_name="s")
    out = jax.ShapeDtypeStruct((_BATCH,), jnp.float32)

    @functools.partial(
        pl.kernel,
        mesh=mesh,
        out_type=(out, out, out),
        scratch_types=[
            pltpu.VMEM((b_per_w,), jnp.int32),
            pltpu.VMEM((b_per_w,), jnp.float32),
            pltpu.VMEM((b_per_w,), jnp.float32),
            pltpu.VMEM((b_per_w,), jnp.float32),
            pltpu.VMEM((b_per_w,), jnp.float32),
            pltpu.VMEM((b_per_w,), jnp.float32),
            pltpu.VMEM((b_per_w,), jnp.float32),
            pltpu.SemaphoreType.DMA,
        ],
    )
    def k(item_hbm, a_hbm, b_hbm, c_hbm, p_hbm, q_hbm, cs_hbm,
          i_idx, a_v, b_v, c_v, p_v, q_v, cs_v, sem):
        wid = lax.axis_index("s") * num_cores + lax.axis_index("c")
        base = wid * b_per_w
        sl_out = pl.ds(base, b_per_w)

        pltpu.sync_copy(item_hbm.at[sl_out], i_idx)
        g_a = pltpu.make_async_copy(a_hbm.at[i_idx], a_v, sem)
        g_b = pltpu.make_async_copy(b_hbm.at[i_idx], b_v, sem)
        g_c = pltpu.make_async_copy(c_hbm.at[i_idx], c_v, sem)
        g_a.start()
        g_b.start()
        g_c.start()
        g_a.wait()
        g_b.wait()
        g_c.wait()

        for i in range(b_per_w // _L):
            sl = pl.ds(i * _L, _L)
            p = -1.73 * _softplus(a_v[sl])
            p_v[sl] = p
            q_v[sl] = 1e-08 - p * b_v[sl]
            cs_v[sl] = 1.0 / (1.0 + jnp.exp(-c_v[sl]))

        cp_p = pltpu.make_async_copy(p_v, p_hbm.at[sl_out], sem)
        cp_q = pltpu.make_async_copy(q_v, q_hbm.at[sl_out], sem)
        cp_c = pltpu.make_async_copy(cs_v, cs_hbm.at[sl_out], sem)
        cp_p.start()
        cp_q.start()
        cp_c.start()
        cp_p.wait()
        cp_q.wait()
        cp_c.wait()

    return k


def _make_theta_kernel(num_cores, b_per_w):
    mesh = plsc.VectorSubcoreMesh(core_axis_name="c", subcore_axis_name="s")

    @functools.partial(
        pl.kernel,
        mesh=mesh,
        out_type=jax.ShapeDtypeStruct((_BATCH,), jnp.float32),
        scratch_types=[
            pltpu.VMEM((b_per_w,), jnp.int32),
            pltpu.VMEM((b_per_w,), jnp.float32),
            pltpu.VMEM((b_per_w,), jnp.float32),
            pltpu.VMEM((b_per_w,), jnp.float32),
            pltpu.VMEM((b_per_w,), jnp.float32),
            pltpu.VMEM((b_per_w,), jnp.float32),
            pltpu.SemaphoreType.DMA,
        ],
    )
    def k(user_hbm, theta_hbm, p_hbm, q_hbm, cs_hbm, out_hbm,
          u_idx, th_v, p_v, q_v, cs_v, o_v, sem):
        wid = lax.axis_index("s") * num_cores + lax.axis_index("c")
        base = wid * b_per_w
        sl_out = pl.ds(base, b_per_w)

        half = b_per_w // 2
        cp_u0 = pltpu.make_async_copy(
            user_hbm.at[pl.ds(base, half)], u_idx.at[pl.ds(0, half)], sem)
        cp_u1 = pltpu.make_async_copy(
            user_hbm.at[pl.ds(base + half, half)],
            u_idx.at[pl.ds(half, half)], sem)
        cp_p = pltpu.make_async_copy(p_hbm.at[sl_out], p_v, sem)
        cp_q = pltpu.make_async_copy(q_hbm.at[sl_out], q_v, sem)
        cp_c = pltpu.make_async_copy(cs_hbm.at[sl_out], cs_v, sem)
        cp_u0.start()
        cp_u1.start()
        cp_p.start()
        cp_q.start()
        cp_c.start()
        cp_u0.wait()
        g_th0 = pltpu.make_async_copy(
            theta_hbm.at[u_idx.at[pl.ds(0, half)]],
            th_v.at[pl.ds(0, half)], sem)
        g_th0.start()
        cp_u1.wait()
        g_th1 = pltpu.make_async_copy(
            theta_hbm.at[u_idx.at[pl.ds(half, half)]],
            th_v.at[pl.ds(half, half)], sem)
        g_th1.start()
        cp_p.wait()
        cp_q.wait()
        cp_c.wait()
        g_th0.wait()

        nc_half = half // _L
        for i in range(nc_half):
            sl = pl.ds(i * _L, _L)
            cs = cs_v[sl]
            z = p_v[sl] * th_v[sl] + q_v[sl]
            o_v[sl] = cs + (1.0 - cs) / (1.0 + jnp.exp(z))

        g_th1.wait()
        for i in range(nc_half, 2 * nc_half):
            sl = pl.ds(i * _L, _L)
            cs = cs_v[sl]
            z = p_v[sl] * th_v[sl] + q_v[sl]
            o_v[sl] = cs + (1.0 - cs) / (1.0 + jnp.exp(z))

        pltpu.sync_copy(o_v, out_hbm.at[sl_out])

    return k


def kernel(user, item, theta_w, a_w, b_w, c_w):
    info = plsc.get_sparse_core_info()
    num_workers = info.num_cores * info.num_subcores
    b_per_w = _BATCH // num_workers
    k_item = _make_item_kernel(info.num_cores, b_per_w)
    k_theta = _make_theta_kernel(info.num_cores, b_per_w)
    p, q, cs = k_item(
        item.astype(jnp.int32),
        _flatten_padded(a_w),
        _flatten_padded(b_w),
        _flatten_padded(c_w),
    )
    return k_theta(
        user.astype(jnp.int32),
        _flatten_padded(theta_w),
        p, q, cs,
    )


# speedup vs baseline: 1.0315x; 1.0196x over previous
import functools

import jax
import jax.numpy as jnp
from jax import lax
from jax.experimental import pallas as pl
from jax.experimental.pallas import tpu as pltpu
from jax.experimental.pallas import tpu_sc as plsc

_BATCH = 16384
_L = 16

_LOG1P_COEFS = (
    0.9999998102178485, -0.4999744938483586, 0.3327617657151469,
    -0.24499611724550963, 0.17757023992299661, -0.10785367917171329,
    0.04421419233802388, -0.008574676204766396,
)


def _log1p_poly(u):
    acc = jnp.float32(_LOG1P_COEFS[-1])
    for c in _LOG1P_COEFS[-2::-1]:
        acc = acc * u + jnp.float32(c)
    return u * acc


def _softplus(x):
    return jnp.maximum(x, 0.0) + _log1p_poly(jnp.exp(-jnp.abs(x)))


def _flatten_padded(w):
    n = w.shape[0]
    n_pad = -n % 1024
    if n_pad:
        w = jnp.pad(w, ((0, n_pad), (0, 0)))
    return w.reshape(-1)


def _make_item_kernel(num_cores, b_per_w):
    mesh = plsc.VectorSubcoreMesh(core_axis_name="c", subcore_axis_name="s")
    out = jax.ShapeDtypeStruct((_BATCH,), jnp.float32)

    @functools.partial(
        pl.kernel,
        mesh=mesh,
        out_type=(out, out, out),
        scratch_types=[
            pltpu.VMEM((b_per_w,), jnp.int32),
            pltpu.VMEM((b_per_w,), jnp.float32),
            pltpu.VMEM((b_per_w,), jnp.float32),
            pltpu.VMEM((b_per_w,), jnp.float32),
            pltpu.VMEM((b_per_w,), jnp.float32),
            pltpu.VMEM((b_per_w,), jnp.float32),
            pltpu.VMEM((b_per_w,), jnp.float32),
            pltpu.SemaphoreType.DMA,
        ],
    )
    def k(item_hbm, a_hbm, b_hbm, c_hbm, p_hbm, q_hbm, cs_hbm,
          i_idx, a_v, b_v, c_v, p_v, q_v, cs_v, sem):
        wid = lax.axis_index("s") * num_cores + lax.axis_index("c")
        base = wid * b_per_w
        sl_out = pl.ds(base, b_per_w)

        pltpu.sync_copy(item_hbm.at[sl_out], i_idx)
        g_a = pltpu.make_async_copy(a_hbm.at[i_idx], a_v, sem)
        g_b = pltpu.make_async_copy(b_hbm.at[i_idx], b_v, sem)
        g_c = pltpu.make_async_copy(c_hbm.at[i_idx], c_v, sem)
        g_a.start()
        g_b.start()
        g_c.start()
        g_a.wait()
        g_b.wait()
        g_c.wait()

        for i in range(b_per_w // _L):
            sl = pl.ds(i * _L, _L)
            p = -1.73 * _softplus(a_v[sl])
            p_v[sl] = p
            q_v[sl] = 1e-08 - p * b_v[sl]
            cs_v[sl] = 1.0 / (1.0 + jnp.exp(-c_v[sl]))

        cp_p = pltpu.make_async_copy(p_v, p_hbm.at[sl_out], sem)
        cp_q = pltpu.make_async_copy(q_v, q_hbm.at[sl_out], sem)
        cp_c = pltpu.make_async_copy(cs_v, cs_hbm.at[sl_out], sem)
        cp_p.start()
        cp_q.start()
        cp_c.start()
        cp_p.wait()
        cp_q.wait()
        cp_c.wait()

    return k


def _make_theta_kernel(num_cores, b_per_w):
    mesh = plsc.VectorSubcoreMesh(core_axis_name="c", subcore_axis_name="s")

    @functools.partial(
        pl.kernel,
        mesh=mesh,
        out_type=jax.ShapeDtypeStruct((_BATCH,), jnp.float32),
        scratch_types=[
            pltpu.VMEM((b_per_w,), jnp.int32),
            pltpu.VMEM((b_per_w,), jnp.float32),
            pltpu.VMEM((b_per_w,), jnp.float32),
            pltpu.VMEM((b_per_w,), jnp.float32),
            pltpu.VMEM((b_per_w,), jnp.float32),
            pltpu.VMEM((b_per_w,), jnp.float32),
            pltpu.SemaphoreType.DMA,
        ],
    )
    def k(user_hbm, theta_hbm, p_hbm, q_hbm, cs_hbm, out_hbm,
          u_idx, th_v, p_v, q_v, cs_v, o_v, sem):
        wid = lax.axis_index("s") * num_cores + lax.axis_index("c")
        base = wid * b_per_w
        sl_out = pl.ds(base, b_per_w)

        cp_u = pltpu.make_async_copy(user_hbm.at[sl_out], u_idx, sem)
        cp_p = pltpu.make_async_copy(p_hbm.at[sl_out], p_v, sem)
        cp_q = pltpu.make_async_copy(q_hbm.at[sl_out], q_v, sem)
        cp_c = pltpu.make_async_copy(cs_hbm.at[sl_out], cs_v, sem)
        cp_u.start()
        cp_p.start()
        cp_q.start()
        cp_c.start()
        cp_u.wait()
        g_th = pltpu.make_async_copy(theta_hbm.at[u_idx], th_v, sem)
        g_th.start()
        cp_p.wait()
        cp_q.wait()
        cp_c.wait()
        g_th.wait()

        for i in range(b_per_w // _L):
            sl = pl.ds(i * _L, _L)
            cs = cs_v[sl]
            z = p_v[sl] * th_v[sl] + q_v[sl]
            o_v[sl] = cs + (1.0 - cs) / (1.0 + jnp.exp(z))

        pltpu.sync_copy(o_v, out_hbm.at[sl_out])

    return k


def kernel(user, item, theta_w, a_w, b_w, c_w):
    info = plsc.get_sparse_core_info()
    num_workers = info.num_cores * info.num_subcores
    b_per_w = _BATCH // num_workers
    k_item = _make_item_kernel(info.num_cores, b_per_w)
    k_theta = _make_theta_kernel(info.num_cores, b_per_w)
    p, q, cs = k_item(
        item.astype(jnp.int32),
        _flatten_padded(a_w),
        _flatten_padded(b_w),
        _flatten_padded(c_w),
    )
    return k_theta(
        user.astype(jnp.int32),
        _flatten_padded(theta_w),
        p, q, cs,
    )
